# Initial kernel scaffold; baseline (speedup 1.0000x reference)
import jax
import jax.numpy as jnp
from jax.experimental import pallas as pl


def kernel(x, keypoints):
    raise NotImplementedError("write your pallas kernel here")


# speedup vs baseline: 452.8018x; 452.8018x over previous
import jax
import jax.numpy as jnp
from jax import lax
from jax.experimental import pallas as pl
from jax.experimental.pallas import tpu as pltpu
from jax.experimental.pallas import tpu_sc as plsc

NC = 2
NS = 16
L = 16
NW = NC * NS

N_TOTAL = 16 * 3 * 512 * 512
PER_W = N_TOTAL // NW
CHUNK = 98304
NCHUNK = PER_W // CHUNK


def _body(x_hbm, kp_hbm, out_hbm, ktab, ctab, stab, buf):
    wid = lax.axis_index("s") * NC + lax.axis_index("c")
    base = wid * PER_W

    pltpu.sync_copy(kp_hbm, ktab.at[pl.ds(0, 32)])
    for j in range(2):
        iv = lax.iota(jnp.int32, L) + (L * j)
        k0 = plsc.load_gather(ktab, [iv])
        k1 = plsc.load_gather(ktab, [jnp.minimum(iv + 1, 31)])
        dk = k1 - k0
        stab[pl.ds(L * j, L)] = dk * 31.0
        ctab[pl.ds(L * j, L)] = k0 - iv.astype(jnp.float32) * dk

    def chunk_body(ci, carry):
        off = base + ci * CHUNK
        pltpu.sync_copy(x_hbm.at[pl.ds(off, CHUNK)], buf)

        def vec_body(v, c2):
            xv = buf[pl.ds(v * L, L)]
            i = jnp.clip((xv * 31.0).astype(jnp.int32), 0, 30)
            c = plsc.load_gather(ctab, [i])
            s = plsc.load_gather(stab, [i])
            buf[pl.ds(v * L, L)] = c + s * xv
            return c2

        lax.fori_loop(0, CHUNK // L, vec_body, 0)
        pltpu.sync_copy(buf, out_hbm.at[pl.ds(off, CHUNK)])
        return carry

    lax.fori_loop(0, NCHUNK, chunk_body, 0)


def kernel(x, keypoints):
    mesh = plsc.VectorSubcoreMesh(
        core_axis_name="c", subcore_axis_name="s", num_cores=NC, num_subcores=NS
    )
    run = pl.kernel(
        _body,
        out_type=jax.ShapeDtypeStruct((N_TOTAL,), jnp.float32),
        mesh=mesh,
        compiler_params=pltpu.CompilerParams(needs_layout_passes=False),
        scratch_types=[
            pltpu.VMEM((128,), jnp.float32),
            pltpu.VMEM((128,), jnp.float32),
            pltpu.VMEM((128,), jnp.float32),
            pltpu.VMEM((CHUNK,), jnp.float32),
        ],
    )
    y = run(x.reshape(-1), keypoints)
    return y.reshape(x.shape)


# speedup vs baseline: 1578.6370x; 3.4864x over previous
import jax
import jax.numpy as jnp
from jax import lax
from jax.experimental import pallas as pl
from jax.experimental.pallas import tpu as pltpu
from jax.experimental.pallas import tpu_sc as plsc

NC = 2
NS = 16
L = 16
NW = NC * NS

N_TOTAL = 16 * 3 * 512 * 512
PER_W = N_TOTAL // NW
CHUNK = 24576
NCHUNK = PER_W // CHUNK
NPAIR = NCHUNK // 2


def _body(x_hbm, kp_hbm, out_hbm, ktab, ctab, stab,
          ibuf0, ibuf1, obuf0, obuf1, isem0, isem1, osem0, osem1):
    ibuf = (ibuf0, ibuf1)
    obuf = (obuf0, obuf1)
    isem = (isem0, isem1)
    osem = (osem0, osem1)

    wid = lax.axis_index("s") * NC + lax.axis_index("c")
    base = wid * PER_W

    pltpu.sync_copy(kp_hbm, ktab.at[pl.ds(0, 32)])
    for j in range(2):
        iv = lax.iota(jnp.int32, L) + (L * j)
        k0 = plsc.load_gather(ktab, [iv])
        k1 = plsc.load_gather(ktab, [jnp.minimum(iv + 1, 31)])
        dk = k1 - k0
        stab[pl.ds(L * j, L)] = dk * 31.0
        ctab[pl.ds(L * j, L)] = k0 - iv.astype(jnp.float32) * dk

    def start_in(g, b):
        return pltpu.async_copy(
            x_hbm.at[pl.ds(base + g * CHUNK, CHUNK)], ibuf[b], isem[b])

    def wait_in(g, b):
        pltpu.make_async_copy(
            x_hbm.at[pl.ds(base + g * CHUNK, CHUNK)], ibuf[b], isem[b]).wait()

    def start_out(g, b):
        return pltpu.async_copy(
            obuf[b], out_hbm.at[pl.ds(base + g * CHUNK, CHUNK)], osem[b])

    def wait_out(g, b):
        pltpu.make_async_copy(
            obuf[b], out_hbm.at[pl.ds(base + g * CHUNK, CHUNK)], osem[b]).wait()

    def compute(b):
        src = ibuf[b]
        dst = obuf[b]

        @plsc.parallel_loop(0, CHUNK, step=L, unroll=8)
        def _(v):
            xv = src[pl.ds(v, L)]
            i = jnp.clip((xv * 31.0).astype(jnp.int32), 0, 30)
            c = plsc.load_gather(ctab, [i])
            s = plsc.load_gather(stab, [i])
            dst[pl.ds(v, L)] = c + s * xv

    start_in(0, 0)
    start_in(1, 1)
    for b in range(2):
        wait_in(b, b)
        compute(b)
        start_out(b, b)
        start_in(b + 2, b)

    def pair_body(t, carry):
        for b in range(2):
            g = 2 * t + b
            wait_out(g - 2, b)
            wait_in(g, b)
            compute(b)
            start_out(g, b)
            start_in(g + 2, b)
        return carry

    lax.fori_loop(1, NPAIR - 1, pair_body, 0)

    for b in range(2):
        g = NCHUNK - 2 + b
        wait_out(g - 2, b)
        wait_in(g, b)
        compute(b)
        start_out(g, b)
    for b in range(2):
        wait_out(NCHUNK - 2 + b, b)


def kernel(x, keypoints):
    mesh = plsc.VectorSubcoreMesh(
        core_axis_name="c", subcore_axis_name="s", num_cores=NC, num_subcores=NS
    )
    run = pl.kernel(
        _body,
        out_type=jax.ShapeDtypeStruct((N_TOTAL,), jnp.float32),
        mesh=mesh,
        compiler_params=pltpu.CompilerParams(needs_layout_passes=False),
        scratch_types=[
            pltpu.VMEM((128,), jnp.float32),
            pltpu.VMEM((128,), jnp.float32),
            pltpu.VMEM((128,), jnp.float32),
            pltpu.VMEM((CHUNK,), jnp.float32),
            pltpu.VMEM((CHUNK,), jnp.float32),
            pltpu.VMEM((CHUNK,), jnp.float32),
            pltpu.VMEM((CHUNK,), jnp.float32),
            pltpu.SemaphoreType.DMA,
            pltpu.SemaphoreType.DMA,
            pltpu.SemaphoreType.DMA,
            pltpu.SemaphoreType.DMA,
        ],
    )
    y = run(x.reshape(-1), keypoints)
    return y.reshape(x.shape)


# speedup vs baseline: 3164.5006x; 2.0046x over previous
import jax
import jax.numpy as jnp
from jax import lax
from jax.experimental import pallas as pl
from jax.experimental.pallas import tpu as pltpu
from jax.experimental.pallas import tpu_sc as plsc

NC = 2
NS = 16
L = 16
NW = NC * NS

ROWS = 16 * 3 * 512
COLS = 512
ROWS_W = ROWS // NW
RCHUNK = 48
NCHUNK = ROWS_W // RCHUNK
NPAIR = NCHUNK // 2
CVEC = COLS // L


def _body(x_hbm, kp_hbm, out_hbm, ktab, ctab, stab,
          ibuf0, ibuf1, obuf0, obuf1, isem0, isem1, osem0, osem1):
    ibuf = (ibuf0, ibuf1)
    obuf = (obuf0, obuf1)
    isem = (isem0, isem1)
    osem = (osem0, osem1)

    wid = lax.axis_index("s") * NC + lax.axis_index("c")
    base = wid * ROWS_W

    pltpu.sync_copy(kp_hbm, ktab.at[pl.ds(0, 32)])
    for j in range(2):
        iv = lax.iota(jnp.int32, L) + (L * j)
        k0 = plsc.load_gather(ktab, [iv])
        k1 = plsc.load_gather(ktab, [jnp.minimum(iv + 1, 31)])
        dk = k1 - k0
        stab[pl.ds(L * j, L)] = dk * 31.0
        ctab[pl.ds(L * j, L)] = k0 - iv.astype(jnp.float32) * dk

    def start_in(g, b):
        return pltpu.async_copy(
            x_hbm.at[pl.ds(base + g * RCHUNK, RCHUNK)], ibuf[b], isem[b])

    def wait_in(g, b):
        pltpu.make_async_copy(
            x_hbm.at[pl.ds(base + g * RCHUNK, RCHUNK)], ibuf[b], isem[b]).wait()

    def start_out(g, b):
        return pltpu.async_copy(
            obuf[b], out_hbm.at[pl.ds(base + g * RCHUNK, RCHUNK)], osem[b])

    def wait_out(g, b):
        pltpu.make_async_copy(
            obuf[b], out_hbm.at[pl.ds(base + g * RCHUNK, RCHUNK)], osem[b]).wait()

    def compute(b):
        src = ibuf[b]
        dst = obuf[b]

        @plsc.parallel_loop(0, RCHUNK * COLS, step=L, unroll=4)
        def _(v):
            r = lax.shift_right_logical(v, 9)
            cc = lax.bitwise_and(v, COLS - 1)
            xv = src[r, pl.ds(cc, L)]
            i = jnp.clip((xv * 31.0).astype(jnp.int32), 0, 30)
            c = plsc.load_gather(ctab, [i])
            s = plsc.load_gather(stab, [i])
            dst[r, pl.ds(cc, L)] = c + s * xv

    start_in(0, 0)
    start_in(1, 1)
    for b in range(2):
        wait_in(b, b)
        compute(b)
        start_out(b, b)
        start_in(b + 2, b)

    def pair_body(t, carry):
        for b in range(2):
            g = 2 * t + b
            wait_out(g - 2, b)
            wait_in(g, b)
            compute(b)
            start_out(g, b)
            start_in(g + 2, b)
        return carry

    lax.fori_loop(1, NPAIR - 1, pair_body, 0)

    for b in range(2):
        g = NCHUNK - 2 + b
        wait_out(g - 2, b)
        wait_in(g, b)
        compute(b)
        start_out(g, b)
    for b in range(2):
        wait_out(NCHUNK - 2 + b, b)


def kernel(x, keypoints):
    mesh = plsc.VectorSubcoreMesh(
        core_axis_name="c", subcore_axis_name="s", num_cores=NC, num_subcores=NS
    )
    run = pl.kernel(
        _body,
        out_type=jax.ShapeDtypeStruct((ROWS, COLS), jnp.float32),
        mesh=mesh,
        compiler_params=pltpu.CompilerParams(needs_layout_passes=False),
        scratch_types=[
            pltpu.VMEM((128,), jnp.float32),
            pltpu.VMEM((128,), jnp.float32),
            pltpu.VMEM((128,), jnp.float32),
            pltpu.VMEM((RCHUNK, COLS), jnp.float32),
            pltpu.VMEM((RCHUNK, COLS), jnp.float32),
            pltpu.VMEM((RCHUNK, COLS), jnp.float32),
            pltpu.VMEM((RCHUNK, COLS), jnp.float32),
            pltpu.SemaphoreType.DMA,
            pltpu.SemaphoreType.DMA,
            pltpu.SemaphoreType.DMA,
            pltpu.SemaphoreType.DMA,
        ],
    )
    y = run(x.reshape(ROWS, COLS), keypoints)
    return y.reshape(x.shape)
